# Initial kernel scaffold; baseline (speedup 1.0000x reference)
import jax
import jax.numpy as jnp
from jax.experimental import pallas as pl


def kernel(x, adj_t, root_ptr, p, batch, group_ptr, Wl1, bl1, Wr1, Wl2, bl2, Wr2, Wlin, blin):
    raise NotImplementedError("write your pallas kernel here")


# speedup vs baseline: 5.4613x; 5.4613x over previous
import functools

import jax
import jax.numpy as jnp
from jax import lax
from jax.experimental import pallas as pl
from jax.experimental.pallas import tpu as pltpu
from jax.experimental.pallas import tpu_sc as plsc

_NC = 2
_NS = 16
_NW = _NC * _NS


def _zero_vmem_rows(ref, nrows, ncols):
  zv = jnp.zeros((16,), jnp.float32)

  def row(i, c):
    for j in range(ncols // 16):
      ref[i, pl.ds(j * 16, 16)] = zv
    return c

  lax.fori_loop(0, nrows, row, 0)


def _fill_vmem_rows(ref, nrows, ncols, val):
  vv = jnp.full((16,), val, jnp.float32)

  def row(i, c):
    for j in range(ncols // 16):
      ref[i, pl.ds(j * 16, 16)] = vv
    return c

  lax.fori_loop(0, nrows, row, 0)


def _copy_rows_chunked(stage, src_sh, dst_hbm, r0, nrows, chunk):
  done = 0
  while done < nrows:
    sz = min(chunk, nrows - done)
    pltpu.sync_copy(src_sh.at[pl.ds(r0 + done, sz)], stage.at[pl.ds(0, sz)])
    pltpu.sync_copy(stage.at[pl.ds(0, sz)], dst_hbm.at[pl.ds(r0 + done, sz)])
    done += sz


def _make_sc_agg(n, e, d, with_deg):
  ch = 128
  epw = e // _NW
  nfull = epw // ch
  tail = epw - nfull * ch
  rpt = (n // _NS) & ~7
  rem_rows = n - _NS * rpt

  mesh = plsc.VectorSubcoreMesh(core_axis_name="c", subcore_axis_name="s")

  out_type = [jax.ShapeDtypeStruct((_NC, n, d), jnp.float32)]
  scratch = [
      pltpu.VMEM((ch,), jnp.int32),
      pltpu.VMEM((ch,), jnp.int32),
      pltpu.VMEM((tail,), jnp.int32),
      pltpu.VMEM((tail,), jnp.int32),
      pltpu.VMEM((ch, d), jnp.float32),
      pltpu.VMEM_SHARED((n, d), jnp.float32),
      pltpu.SemaphoreType.DMA,
  ]
  if with_deg:
    out_type.append(jax.ShapeDtypeStruct((_NC * n,), jnp.float32))
    scratch.append(pltpu.VMEM((rpt,), jnp.float32))

  def zero_acc(rows, acc_sh, sid, r0):
    done = 0
    while done < rpt:
      sz = min(ch, rpt - done)
      pltpu.sync_copy(rows.at[pl.ds(0, sz)], acc_sh.at[pl.ds(r0 + done, sz)])
      done += sz
    if rem_rows:
      @pl.when(sid == 0)
      def _():
        pltpu.sync_copy(rows.at[pl.ds(0, rem_rows)],
                        acc_sh.at[pl.ds(_NS * rpt, rem_rows)])

  def body(*refs):
    if with_deg:
      (src_h, dst_h, x_h, acc_o, deg_o,
       sidx, didx, sidx_t, didx_t, rows, acc_sh, sem, outst) = refs
    else:
      (src_h, dst_h, x_h, acc_o,
       sidx, didx, sidx_t, didx_t, rows, acc_sh, sem) = refs

    cid = lax.axis_index("c")
    sid = lax.axis_index("s")
    wid = sid * _NC + cid
    r0 = pl.multiple_of(sid * rpt, 8)
    base = wid * epw

    _zero_vmem_rows(rows, ch, d)
    zero_acc(rows, acc_sh, sid, r0)
    plsc.subcore_barrier()

    def chunk_body(i, c):
      off = pl.multiple_of(base + i * ch, 8)
      pltpu.sync_copy(src_h.at[pl.ds(off, ch)], sidx)
      pltpu.sync_copy(dst_h.at[pl.ds(off, ch)], didx)
      pltpu.async_copy(x_h.at[sidx], rows, sem).wait()
      pltpu.sync_copy(rows, acc_sh.at[didx], add=True)
      return c

    lax.fori_loop(0, nfull, chunk_body, 0)

    if tail:
      offt = pl.multiple_of(base + nfull * ch, 8)
      pltpu.sync_copy(src_h.at[pl.ds(offt, tail)], sidx_t)
      pltpu.sync_copy(dst_h.at[pl.ds(offt, tail)], didx_t)
      pltpu.async_copy(x_h.at[sidx_t], rows.at[pl.ds(0, tail)], sem).wait()
      pltpu.sync_copy(rows.at[pl.ds(0, tail)], acc_sh.at[didx_t], add=True)

    plsc.subcore_barrier()

    _copy_rows_chunked(rows, acc_sh, acc_o.at[cid], r0, rpt, ch)
    if rem_rows:
      @pl.when(sid == 0)
      def _():
        _copy_rows_chunked(rows, acc_sh, acc_o.at[cid], _NS * rpt,
                           rem_rows, ch)

    if not with_deg:
      return

    plsc.subcore_barrier()
    _zero_vmem_rows(rows, ch, d)
    zero_acc(rows, acc_sh, sid, r0)
    _fill_vmem_rows(rows, ch, d, 1.0)
    plsc.subcore_barrier()

    def deg_chunk(i, c):
      off = pl.multiple_of(base + i * ch, 8)
      pltpu.sync_copy(dst_h.at[pl.ds(off, ch)], didx)
      pltpu.sync_copy(rows, acc_sh.at[didx], add=True)
      return c

    lax.fori_loop(0, nfull, deg_chunk, 0)
    if tail:
      offt = pl.multiple_of(base + nfull * ch, 8)
      pltpu.sync_copy(dst_h.at[pl.ds(offt, tail)], didx_t)
      pltpu.sync_copy(rows.at[pl.ds(0, tail)], acc_sh.at[didx_t], add=True)
    plsc.subcore_barrier()

    lane = lax.iota(jnp.int32, 16)
    done = 0
    while done < rpt:
      sz = min(ch, rpt - done)
      pltpu.sync_copy(acc_sh.at[pl.ds(r0 + done, sz)], rows.at[pl.ds(0, sz)])
      dbase = done

      def ext(k, c):
        acc = jnp.zeros((16,), jnp.float32)
        for j in range(16):
          rv = rows[k * 16 + j, pl.ds(0, 16)]
          acc = jnp.where(lane == j, rv, acc)
        outst[pl.ds(dbase + k * 16, 16)] = acc
        return c

      lax.fori_loop(0, sz // 16, ext, 0)
      done += sz
    pltpu.sync_copy(outst,
                    deg_o.at[pl.ds(pl.multiple_of(cid * n + r0, 8), rpt)])
    if rem_rows:
      @pl.when(sid == 0)
      def _():
        pltpu.sync_copy(acc_sh.at[pl.ds(_NS * rpt, rem_rows)],
                        rows.at[pl.ds(0, rem_rows)])
        acc2 = jnp.zeros((16,), jnp.float32)
        for j in range(16):
          rv2 = rows[j, pl.ds(0, 16)]
          acc2 = jnp.where(lane == j, rv2, acc2)
        outst[pl.ds(0, 16)] = acc2
        pltpu.sync_copy(
            outst.at[pl.ds(0, rem_rows)],
            deg_o.at[pl.ds(pl.multiple_of(cid * n + _NS * rpt, 8),
                           rem_rows)])

  return pl.kernel(body, out_type=out_type, mesh=mesh, scratch_types=scratch)


def _c1(a, b):
  return lax.dot_general(a, b, (((1,), (1,)), ((), ())),
                         preferred_element_type=jnp.float32)


def _c0(a, b):
  return lax.dot_general(a, b, (((0,), (0,)), ((), ())),
                         preferred_element_type=jnp.float32)


def _tc_layer1(acc0, acc1, dg0, dg1, x, p2, wl, bl2, wr, n, d, bn):
  nblk = n // bn

  def body(a0, a1, g0, g1, xr, pr, wlr, blr, wrr, o):
    deg = jnp.maximum(g0[...] + g1[...], 1.0)
    agg = (a0[...] + a1[...]) / deg
    h = _c1(agg, wlr[...]) + blr[...] + _c1(xr[...], wrr[...])
    o[...] = jnp.maximum(h, 0.0) * pr[...]

  row = pl.BlockSpec((bn, d), lambda i: (i, 0))
  col1 = pl.BlockSpec((bn, 1), lambda i: (i, 0))
  wspec = pl.BlockSpec((d, d), lambda i: (0, 0))
  bspec = pl.BlockSpec((1, d), lambda i: (0, 0))
  return pl.pallas_call(
      body,
      grid=(nblk,),
      in_specs=[row, row, col1, col1, row, col1, wspec, bspec, wspec],
      out_specs=row,
      out_shape=jax.ShapeDtypeStruct((n, d), jnp.float32),
  )(acc0, acc1, dg0, dg1, x, p2, wl, bl2, wr)


def _tc_final(acc0, acc1, dg0, dg1, xs, p2, bat2, root2, wl, bl2, wr,
              wa, wb, blin2, n, d, g, bn):
  nblk = n // bn

  def body(a0, a1, g0, g1, xr, pr, br, rr, wlr, blr, wrr, war, wbr, bor,
           o, pool, hroot, cnt):
    i = pl.program_id(0)

    @pl.when(i == 0)
    def _():
      pool[...] = jnp.zeros_like(pool)
      hroot[...] = jnp.zeros_like(hroot)
      cnt[...] = jnp.zeros_like(cnt)

    deg = jnp.maximum(g0[...] + g1[...], 1.0)
    agg = (a0[...] + a1[...]) / deg
    h2 = jnp.maximum(_c1(agg, wlr[...]) + blr[...] + _c1(xr[...], wrr[...]),
                     0.0)
    hp = h2 * pr[...]

    gid_row = lax.broadcasted_iota(jnp.int32, (1, g), 1)
    st = (br[...] == gid_row).astype(jnp.float32)
    pool[...] += _c0(st, hp)
    cnt[...] += _c0(st, jnp.ones((bn, 1), jnp.float32))

    pos = i * bn + lax.broadcasted_iota(jnp.int32, (bn, 1), 0)
    rt = (pos == rr[...]).astype(jnp.float32)
    hroot[...] += _c0(rt, h2)

    @pl.when(i == nblk - 1)
    def _():
      pooled = pool[...] / jnp.maximum(cnt[...], 1.0)
      o[...] = _c1(hroot[...], war[...]) + _c1(pooled, wbr[...]) + bor[...]

  row = pl.BlockSpec((bn, d), lambda i: (i, 0))
  col1 = pl.BlockSpec((bn, 1), lambda i: (i, 0))
  wspec = pl.BlockSpec((d, d), lambda i: (0, 0))
  bspec = pl.BlockSpec((1, d), lambda i: (0, 0))
  gspec = pl.BlockSpec((1, g), lambda i: (0, 0))
  ospec = pl.BlockSpec((g, d), lambda i: (0, 0))
  return pl.pallas_call(
      body,
      grid=(nblk,),
      in_specs=[row, row, col1, col1, row, col1, col1, gspec,
                wspec, bspec, wspec, wspec, wspec, bspec],
      out_specs=ospec,
      out_shape=jax.ShapeDtypeStruct((g, d), jnp.float32),
      scratch_shapes=[
          pltpu.VMEM((g, d), jnp.float32),
          pltpu.VMEM((g, d), jnp.float32),
          pltpu.VMEM((g, 1), jnp.float32),
      ],
  )(acc0, acc1, dg0, dg1, xs, p2, bat2, root2, wl, bl2, wr, wa, wb, blin2)


def kernel(x, adj_t, root_ptr, p, batch, group_ptr,
           Wl1, bl1, Wr1, Wl2, bl2, Wr2, Wlin, blin):
  del group_ptr
  n, d = x.shape
  e = adj_t.shape[1]
  g = root_ptr.shape[0]
  bn = 1000

  src = adj_t[0].astype(jnp.int32)
  dst = adj_t[1].astype(jnp.int32)
  p2 = p.reshape(n, 1)
  bat2 = batch.astype(jnp.int32).reshape(n, 1)
  root2 = root_ptr.astype(jnp.int32).reshape(1, g)
  bl1_2 = bl1.reshape(1, d)
  bl2_2 = bl2.reshape(1, d)
  blin2 = blin.reshape(1, d)
  wa = Wlin[:, :d]
  wb = Wlin[:, d:]

  sc_agg_deg = _make_sc_agg(n, e, d, with_deg=True)
  sc_agg = _make_sc_agg(n, e, d, with_deg=False)

  acc1, degf = sc_agg_deg(src, dst, x)
  dg0 = degf[:n].reshape(n, 1)
  dg1 = degf[n:].reshape(n, 1)
  xs = _tc_layer1(acc1[0], acc1[1], dg0, dg1, x, p2,
                  Wl1, bl1_2, Wr1, n, d, bn)
  acc2 = sc_agg(src, dst, xs)
  if isinstance(acc2, (list, tuple)):
    acc2 = acc2[0]
  return _tc_final(acc2[0], acc2[1], dg0, dg1, xs, p2, bat2, root2,
                   Wl2, bl2_2, Wr2, wa, wb, blin2, n, d, g, bn)
